# Initial kernel scaffold; baseline (speedup 1.0000x reference)
import jax
import jax.numpy as jnp
from jax.experimental import pallas as pl


def kernel(keys, table):
    raise NotImplementedError("write your pallas kernel here")


# speedup vs baseline: 1.9397x; 1.9397x over previous
import jax
import jax.numpy as jnp
from jax import lax
from jax.experimental import pallas as pl
from jax.experimental.pallas import tpu as pltpu
from jax.experimental.pallas import tpu_sc as plsc

TABLE_SIZE = 262144
MASK = TABLE_SIZE - 1
PRIMES = (6700417, 15485863, 32452843, 49979687)
NUM_HEADS = 4
D = 128
L = 16
NC, NS = 2, 16
NW = NC * NS
N_KEYS = 4096 * 64
KEYS_PER_W = N_KEYS // NW
CHUNK = 64
N_CHUNKS = KEYS_PER_W // CHUNK


def _sc_body(keys_hbm, table_hbm, out_hbm, keys_v, idx_v, rows_v, out_v, sem):
    si = lax.axis_index("s")
    ci = lax.axis_index("c")

    def _chunk(c, _):
        pltpu.sync_copy(keys_hbm.at[si, ci, c], keys_v)
        for v in range(CHUNK // L):
            sl = pl.ds(v * L, L)
            k = keys_v[sl]
            for j in range(NUM_HEADS):
                idx_v[j, sl] = (k * jnp.int32(PRIMES[j])) & jnp.int32(MASK)
        copies = [
            pltpu.async_copy(
                table_hbm.at[idx_v.at[jnp.int32(j)]], rows_v.at[jnp.int32(j)], sem
            )
            for j in range(NUM_HEADS)
        ]
        for cp in copies:
            cp.wait()

        def _key(i, _):
            for cc in range(D // L):
                sl = pl.ds(cc * L, L)
                acc = (rows_v[0, i, sl] + rows_v[1, i, sl]) + (
                    rows_v[2, i, sl] + rows_v[3, i, sl]
                )
                out_v[i, sl] = acc * jnp.float32(0.25)
            return _

        lax.fori_loop(jnp.int32(0), jnp.int32(CHUNK), _key, 0)

        pltpu.sync_copy(out_v, out_hbm.at[si, ci, c])
        return _

    lax.fori_loop(jnp.int32(0), jnp.int32(N_CHUNKS), _chunk, 0)


@jax.jit
def _sc_lookup(keys_grouped, table):
    mesh = plsc.VectorSubcoreMesh(
        core_axis_name="c", subcore_axis_name="s", num_cores=NC, num_subcores=NS
    )
    f = pl.kernel(
        _sc_body,
        out_type=jax.ShapeDtypeStruct((NS, NC, N_CHUNKS, CHUNK, D), jnp.float32),
        mesh=mesh,
        scratch_types=[
            pltpu.VMEM((CHUNK,), jnp.int32),
            pltpu.VMEM((NUM_HEADS, CHUNK), jnp.int32),
            pltpu.VMEM((NUM_HEADS, CHUNK, D), jnp.float32),
            pltpu.VMEM((CHUNK, D), jnp.float32),
            pltpu.SemaphoreType.DMA,
        ],
    )
    return f(keys_grouped, table)


def kernel(keys, table):
    M, G = keys.shape
    keys_grouped = keys.reshape(NS, NC, N_CHUNKS, CHUNK).astype(jnp.int32)
    out = _sc_lookup(keys_grouped, table)
    return out.reshape(M, G, table.shape[1])


# speedup vs baseline: 4.0437x; 2.0847x over previous
import jax
import jax.numpy as jnp
from jax import lax
from jax.experimental import pallas as pl
from jax.experimental.pallas import tpu as pltpu
from jax.experimental.pallas import tpu_sc as plsc

TABLE_SIZE = 262144
MASK = TABLE_SIZE - 1
PRIMES = (6700417, 15485863, 32452843, 49979687)
NUM_HEADS = 4
D = 128
L = 16
NC, NS = 2, 16
N_KEYS = 4096 * 64
KEYS_PER_W = N_KEYS // (NC * NS)
CHUNK = 64
N_CHUNKS = KEYS_PER_W // CHUNK
N_PAIRS = N_CHUNKS // 2


def _sc_body(keys_hbm, table_hbm, out_hbm, keys_v, idx_v, rows_v, out_v, sems):
    si = lax.axis_index("s")
    ci = lax.axis_index("c")
    gsem = [sems.at[jnp.int32(0)], sems.at[jnp.int32(1)]]
    wsem = [sems.at[jnp.int32(2)], sems.at[jnp.int32(3)]]

    pltpu.sync_copy(keys_hbm.at[si, ci], keys_v)

    def _hash_and_issue(g, buf):
        off = g * jnp.int32(CHUNK)
        for v in range(CHUNK // L):
            k = keys_v[pl.ds(off + jnp.int32(v * L), L)]
            sl = pl.ds(v * L, L)
            for j in range(NUM_HEADS):
                idx_v[buf, j, sl] = (k * jnp.int32(PRIMES[j])) & jnp.int32(MASK)
        for j in range(NUM_HEADS):
            pltpu.async_copy(
                table_hbm.at[idx_v.at[jnp.int32(buf), jnp.int32(j)]],
                rows_v.at[jnp.int32(buf), jnp.int32(j)],
                gsem[buf],
            )

    def _wait_gathers(buf):
        for j in range(NUM_HEADS):
            pltpu.make_async_copy(
                table_hbm.at[idx_v.at[jnp.int32(buf), jnp.int32(j)]],
                rows_v.at[jnp.int32(buf), jnp.int32(j)],
                gsem[buf],
            ).wait()

    def _accumulate(buf):
        def _key(i, carry):
            for cc in range(D // L):
                sl = pl.ds(cc * L, L)
                acc = (rows_v[buf, 0, i, sl] + rows_v[buf, 1, i, sl]) + (
                    rows_v[buf, 2, i, sl] + rows_v[buf, 3, i, sl]
                )
                out_v[buf, i, sl] = acc * jnp.float32(0.25)
            return carry

        lax.fori_loop(jnp.int32(0), jnp.int32(CHUNK), _key, 0)

    def _wait_write(buf):
        pltpu.make_async_copy(
            out_v.at[jnp.int32(buf)], out_hbm.at[si, ci, jnp.int32(buf)], wsem[buf]
        ).wait()

    def _start_write(g, buf):
        pltpu.async_copy(
            out_v.at[jnp.int32(buf)], out_hbm.at[si, ci, g], wsem[buf]
        )

    _hash_and_issue(jnp.int32(0), 0)

    def _pair(c2, carry):
        g0 = c2 * jnp.int32(2)
        g1 = g0 + jnp.int32(1)

        _hash_and_issue(g1, 1)
        _wait_gathers(0)

        @pl.when(c2 > jnp.int32(0))
        def _():
            _wait_write(0)

        _accumulate(0)
        _start_write(g0, 0)

        @pl.when(c2 < jnp.int32(N_PAIRS - 1))
        def _():
            _hash_and_issue(g0 + jnp.int32(2), 0)

        _wait_gathers(1)

        @pl.when(c2 > jnp.int32(0))
        def _():
            _wait_write(1)

        _accumulate(1)
        _start_write(g1, 1)
        return carry

    lax.fori_loop(jnp.int32(0), jnp.int32(N_PAIRS), _pair, 0)
    _wait_write(0)
    _wait_write(1)


@jax.jit
def _sc_lookup(keys_grouped, table):
    mesh = plsc.VectorSubcoreMesh(
        core_axis_name="c", subcore_axis_name="s", num_cores=NC, num_subcores=NS
    )
    f = pl.kernel(
        _sc_body,
        out_type=jax.ShapeDtypeStruct((NS, NC, N_CHUNKS, CHUNK, D), jnp.float32),
        mesh=mesh,
        scratch_types=[
            pltpu.VMEM((KEYS_PER_W,), jnp.int32),
            pltpu.VMEM((2, NUM_HEADS, CHUNK), jnp.int32),
            pltpu.VMEM((2, NUM_HEADS, CHUNK, D), jnp.float32),
            pltpu.VMEM((2, CHUNK, D), jnp.float32),
            pltpu.SemaphoreType.DMA((4,)),
        ],
    )
    return f(keys_grouped, table)


def kernel(keys, table):
    M, G = keys.shape
    keys_grouped = keys.reshape(NS, NC, KEYS_PER_W).astype(jnp.int32)
    out = _sc_lookup(keys_grouped, table)
    return out.reshape(M, G, table.shape[1])
